# baseline (speedup 1.0000x reference)
import jax
import jax.numpy as jnp
from jax import lax
from jax.experimental import pallas as pl

_N = 5000
_B = 128
_NB = 40
_NPAD = _NB * _B
_THR = 0.5


def _nms_body(x1c, y1c, x2c, y2c, x1r, y1r, x2r, y2r, keep_ref, area_ref):
    keep_ref[...] = jnp.ones((_NB, _B), jnp.float32)
    area_ref[...] = (x2r[...] - x1r[...]) * (y2r[...] - y1r[...])

    riota = lax.broadcasted_iota(jnp.int32, (_B, _B), 0)
    ciota = lax.broadcasted_iota(jnp.int32, (_B, _B), 1)
    tri = riota < ciota
    ident = riota == ciota

    def iou_chunk(bx1, by1, bx2, by2, area_b, c):
        ax1 = x1r[pl.ds(c, 1), :]
        ay1 = y1r[pl.ds(c, 1), :]
        ax2 = x2r[pl.ds(c, 1), :]
        ay2 = y2r[pl.ds(c, 1), :]
        area_a = area_ref[pl.ds(c, 1), :]
        xx1 = jnp.maximum(bx1, ax1)
        yy1 = jnp.maximum(by1, ay1)
        xx2 = jnp.minimum(bx2, ax2)
        yy2 = jnp.minimum(by2, ay2)
        inter = jnp.maximum(xx2 - xx1, 0.0) * jnp.maximum(yy2 - yy1, 0.0)
        union = area_b + area_a - inter
        return inter / jnp.maximum(union, 1e-12)

    def block_body(k, _):
        base = k * _B
        bx1 = x1c[pl.ds(base, _B), :]
        by1 = y1c[pl.ds(base, _B), :]
        bx2 = x2c[pl.ds(base, _B), :]
        by2 = y2c[pl.ds(base, _B), :]
        area_b = (bx2 - bx1) * (by2 - by1)

        iou_bb = iou_chunk(bx1, by1, bx2, by2, area_b, k)
        s_bb = jnp.where((iou_bb > _THR) & tri, 1.0, 0.0).astype(jnp.bfloat16)
        ext = keep_ref[pl.ds(k, 1), :]

        def fix_cond(carry):
            return carry[1]

        def fix_body(carry):
            kp, _ = carry
            sup = lax.dot_general(
                kp.astype(jnp.bfloat16), s_bb,
                (((1,), (0,)), ((), ())),
                preferred_element_type=jnp.float32,
            )
            new = jnp.where(sup > 0.0, 0.0, ext)
            changed = jnp.any(new != kp)
            return (new, changed)

        keep_blk, _ = lax.while_loop(fix_cond, fix_body, (ext, True))
        keep_ref[pl.ds(k, 1), :] = keep_blk

        kc = jnp.any(ident & (keep_blk > 0.0), axis=1, keepdims=True)

        def tail_body(i, _):
            c1 = k + 1 + 2 * i
            c2 = c1 + 1
            c2c = jnp.minimum(c2, _NB - 1)
            iou1 = iou_chunk(bx1, by1, bx2, by2, area_b, c1)
            iou2 = iou_chunk(bx1, by1, bx2, by2, area_b, c2c)
            sup1 = jnp.any((iou1 > _THR) & kc, axis=0, keepdims=True)
            sup2 = jnp.any((iou2 > _THR) & kc, axis=0, keepdims=True)
            cur1 = keep_ref[pl.ds(c1, 1), :]
            keep_ref[pl.ds(c1, 1), :] = jnp.where(sup1, 0.0, cur1)
            cur2 = keep_ref[pl.ds(c2c, 1), :]
            keep_ref[pl.ds(c2c, 1), :] = jnp.where(sup2 & (c2 < _NB), 0.0, cur2)
            return 0

        lax.fori_loop(0, (_NB - k) // 2, tail_body, 0)
        return 0

    lax.fori_loop(0, _NB, block_body, 0)


def kernel(boxes, scores):
    order = jnp.argsort(-scores)
    b = boxes[order]
    pad = jnp.zeros((_NPAD - _N, 4), jnp.float32)
    bp = jnp.concatenate([b, pad], axis=0)

    cols = [bp[:, i : i + 1] for i in range(4)]
    rows = [bp[:, i].reshape(_NB, _B) for i in range(4)]

    keep_pad, _ = pl.pallas_call(
        _nms_body,
        out_shape=[
            jax.ShapeDtypeStruct((_NB, _B), jnp.float32),
            jax.ShapeDtypeStruct((_NB, _B), jnp.float32),
        ],
    )(*cols, *rows)

    keep_sorted = keep_pad.reshape(_NPAD)[:_N]
    m = jnp.zeros((_N,), jnp.float32).at[order].set(keep_sorted)
    out = jnp.concatenate([boxes * m[:, None], (scores * m)[:, None]], axis=1)
    return out


# speedup vs baseline: 1.0474x; 1.0474x over previous
import jax
import jax.numpy as jnp
from jax import lax
from jax.experimental import pallas as pl
from jax.experimental.pallas import tpu as pltpu

_N = 5000
_B = 128
_NB = 40
_NPAD = _NB * _B
_THR = 0.5


def _nms_body(x1c, y1c, x2c, y2c, x1r, y1r, x2r, y2r, keep_ref, area_ref,
              keep8_ref):
    keep_ref[...] = jnp.ones((_NB, _B), jnp.float32)
    keep8_ref[...] = jnp.ones((_NB * 8, _B), jnp.float32)
    area_ref[...] = (x2r[...] - x1r[...]) * (y2r[...] - y1r[...])

    riota = lax.broadcasted_iota(jnp.int32, (_B, _B), 0)
    ciota = lax.broadcasted_iota(jnp.int32, (_B, _B), 1)
    tri = riota < ciota
    ident = riota == ciota

    def iou_chunk(bx1, by1, bx2, by2, area_b, c):
        ax1 = x1r[pl.ds(c, 1), :]
        ay1 = y1r[pl.ds(c, 1), :]
        ax2 = x2r[pl.ds(c, 1), :]
        ay2 = y2r[pl.ds(c, 1), :]
        area_a = area_ref[pl.ds(c, 1), :]
        xx1 = jnp.maximum(bx1, ax1)
        yy1 = jnp.maximum(by1, ay1)
        xx2 = jnp.minimum(bx2, ax2)
        yy2 = jnp.minimum(by2, ay2)
        inter = jnp.maximum(xx2 - xx1, 0.0) * jnp.maximum(yy2 - yy1, 0.0)
        union = area_b + area_a - inter
        return inter / jnp.maximum(union, 1e-12)

    def block_body(k, _):
        base = k * _B
        bx1 = x1c[pl.ds(base, _B), :]
        by1 = y1c[pl.ds(base, _B), :]
        bx2 = x2c[pl.ds(base, _B), :]
        by2 = y2c[pl.ds(base, _B), :]
        area_b = (bx2 - bx1) * (by2 - by1)

        iou_bb = iou_chunk(bx1, by1, bx2, by2, area_b, k)
        s_bb = jnp.where((iou_bb > _THR) & tri, 1.0, 0.0).astype(jnp.bfloat16)
        ext8 = keep8_ref[pl.ds(8 * k, 8), :]
        ext = jnp.all(ext8 > 0.0, axis=0, keepdims=True).astype(jnp.float32)

        def fix_cond(carry):
            return carry[1]

        def fix_body(carry):
            kp, _ = carry
            sup = lax.dot_general(
                kp.astype(jnp.bfloat16), s_bb,
                (((1,), (0,)), ((), ())),
                preferred_element_type=jnp.float32,
            )
            new = jnp.where(sup > 0.0, 0.0, ext)
            changed = jnp.any(new != kp)
            return (new, changed)

        keep_blk, _ = lax.while_loop(fix_cond, fix_body, (ext, True))
        keep_ref[pl.ds(k, 1), :] = keep_blk

        kc = jnp.any(ident & (keep_blk > 0.0), axis=1, keepdims=True)

        def sup8_of(iou_c):
            m3 = jnp.reshape((iou_c > _THR) & kc, (16, 8, _B))
            return jnp.any(m3, axis=0)

        def tail_body(i, _):
            c1 = k + 1 + 2 * i
            c2 = c1 + 1
            c2c = jnp.minimum(c2, _NB - 1)
            iou1 = iou_chunk(bx1, by1, bx2, by2, area_b, c1)
            iou2 = iou_chunk(bx1, by1, bx2, by2, area_b, c2c)
            sup1 = sup8_of(iou1)
            sup2 = sup8_of(iou2)
            cur1 = keep8_ref[pl.ds(8 * c1, 8), :]
            keep8_ref[pl.ds(8 * c1, 8), :] = jnp.where(sup1, 0.0, cur1)
            cur2 = keep8_ref[pl.ds(8 * c2c, 8), :]
            keep8_ref[pl.ds(8 * c2c, 8), :] = jnp.where(
                sup2 & (c2 < _NB), 0.0, cur2
            )
            return 0

        lax.fori_loop(0, (_NB - k) // 2, tail_body, 0)
        return 0

    lax.fori_loop(0, _NB, block_body, 0)


def kernel(boxes, scores):
    order = jnp.argsort(-scores)
    b = boxes[order]
    pad = jnp.zeros((_NPAD - _N, 4), jnp.float32)
    bp = jnp.concatenate([b, pad], axis=0)

    cols = [bp[:, i : i + 1] for i in range(4)]
    rows = [bp[:, i].reshape(_NB, _B) for i in range(4)]

    keep_pad, _ = pl.pallas_call(
        _nms_body,
        out_shape=[
            jax.ShapeDtypeStruct((_NB, _B), jnp.float32),
            jax.ShapeDtypeStruct((_NB, _B), jnp.float32),
        ],
        scratch_shapes=[pltpu.VMEM((_NB * 8, _B), jnp.float32)],
    )(*cols, *rows)

    keep_sorted = keep_pad.reshape(_NPAD)[:_N]
    m = jnp.zeros((_N,), jnp.float32).at[order].set(keep_sorted)
    out = jnp.concatenate([boxes * m[:, None], (scores * m)[:, None]], axis=1)
    return out
